# baseline (speedup 1.0000x reference)
import jax
import jax.numpy as jnp
from jax import lax
from jax.experimental import pallas as pl
from jax.experimental.pallas import tpu as pltpu

_NB = 1024
_NBUF = 4
_RSPLIT = 8


def _trunk_body(xs_ref, xm_ref, w1_ref, b1_ref, w2_ref, b2_ref, wld_ref,
                hcat_ref, p_ref):
    emb = xs_ref.shape[1]
    hid = w2_ref.shape[0]
    w1 = w1_ref[...]
    h = lax.dot_general(xs_ref[...], w1[:, :emb], (((1,), (1,)), ((), ())),
                        preferred_element_type=jnp.float32)
    h = h + lax.dot_general(xm_ref[...], w1[:, emb:], (((1,), (1,)), ((), ())),
                            preferred_element_type=jnp.float32)
    h = jnp.maximum(h + b1_ref[...], 0.0)
    h = lax.dot_general(h, w2_ref[...], (((1,), (1,)), ((), ())),
                        preferred_element_type=jnp.float32)
    h = jnp.maximum(h + b2_ref[...], 0.0)
    hb = h.astype(jnp.bfloat16)
    wld = wld_ref[...]
    nworlds = p_ref.shape[1]
    for w in range(nworlds):
        hcat_ref[:, w * hid:(w + 1) * hid] = jnp.where(wld == w, hb, 0)
    widx = lax.broadcasted_iota(jnp.int32, (wld.shape[0], nworlds), 1)
    p_ref[...] = (wld == widx).astype(jnp.bfloat16)


def _make_head_body(nb, n_full):
    def body(hcat_ref, bh_ref, p_ref, wh_hbm, out_hbm,
             wh_buf, acc_buf, sem_in, sem_out):
        nworlds = wh_hbm.shape[0]
        b = hcat_ref.shape[0]
        rs = b // _RSPLIT
        n = pl.program_id(0)
        nsteps = pl.num_programs(0)

        def read_copies(blk_idx, width):
            buf_slot = lax.rem(blk_idx, _NBUF)
            return [pltpu.make_async_copy(
                wh_hbm.at[w, pl.ds(blk_idx * nb, width), :],
                wh_buf.at[buf_slot, w, pl.ds(0, width), :],
                sem_in.at[buf_slot, w]) for w in range(nworlds)]

        def write_copies(blk_idx, width):
            buf_slot = lax.rem(blk_idx, _NBUF)
            return [pltpu.make_async_copy(
                acc_buf.at[buf_slot, pl.ds(r * rs, rs), pl.ds(0, width)],
                out_hbm.at[pl.ds(r * rs, rs), pl.ds(blk_idx * nb, width)],
                sem_out.at[buf_slot, r]) for r in range(_RSPLIT)]

        @pl.when(n == 0)
        def _():
            for m in range(min(_NBUF - 1, n_full)):
                for c in read_copies(m, nb):
                    c.start()

        @pl.when(n + _NBUF - 1 < n_full)
        def _():
            for c in read_copies(n + _NBUF - 1, nb):
                c.start()

        @pl.when(n >= _NBUF)
        def _():
            for c in write_copies(n - _NBUF, nb):
                c.wait()

        for c in read_copies(n, nb):
            c.wait()

        acc = lax.dot_general(p_ref[...], bh_ref[...].astype(jnp.bfloat16),
                              (((1,), (0,)), ((), ())),
                              preferred_element_type=jnp.float32)
        slot = lax.rem(n, _NBUF)
        whcat = jnp.concatenate(
            [wh_buf[slot, w].astype(jnp.bfloat16) for w in range(nworlds)],
            axis=-1)
        acc = acc + lax.dot_general(hcat_ref[...], whcat,
                                    (((1,), (1,)), ((), ())),
                                    preferred_element_type=jnp.float32)
        acc_buf[slot] = acc

        for c in write_copies(n, nb):
            c.start()

        @pl.when(n == nsteps - 1)
        def _():
            for k in range(_NBUF - 1, -1, -1):
                @pl.when(n - k >= 0)
                def _(k=k):
                    for c in write_copies(n - k, nb):
                        c.wait()

    return body


def _tail_body(hcat_ref, wh_ref, bh_ref, p_ref, prev_ref, out_ref):
    nworlds = wh_ref.shape[0]
    acc = lax.dot_general(p_ref[...], bh_ref[...].astype(jnp.bfloat16),
                          (((1,), (0,)), ((), ())),
                          preferred_element_type=jnp.float32)
    whcat = jnp.concatenate(
        [wh_ref[w].astype(jnp.bfloat16) for w in range(nworlds)], axis=-1)
    acc = acc + lax.dot_general(hcat_ref[...], whcat, (((1,), (1,)), ((), ())),
                                preferred_element_type=jnp.float32)
    out_ref[...] = acc


def kernel(source, mode, context_id, se, me, W1, b1, W2, b2, Wh, bh):
    B = source.shape[0]
    HID = W1.shape[0]
    NW, N, _ = Wh.shape

    xs = jnp.take(se, source, axis=0)
    xm = jnp.take(me, mode, axis=0)
    worlds = jnp.clip(context_id.astype(jnp.int32) - 1, 0, NW - 1)
    worlds = worlds.reshape(B, 1)

    hcat, p = pl.pallas_call(
        _trunk_body,
        out_shape=(jax.ShapeDtypeStruct((B, NW * HID), jnp.bfloat16),
                   jax.ShapeDtypeStruct((B, NW), jnp.bfloat16)),
    )(xs, xm, W1, b1.reshape(1, HID), W2, b2.reshape(1, HID), worlds)

    nb = _NB
    n_full = N // nb
    tail = N - n_full * nb
    out = pl.pallas_call(
        _make_head_body(nb, n_full),
        grid=(n_full,),
        in_specs=[
            pl.BlockSpec((B, NW * HID), lambda n: (0, 0)),
            pl.BlockSpec((NW, nb), lambda n: (0, n)),
            pl.BlockSpec((B, NW), lambda n: (0, 0)),
            pl.BlockSpec(memory_space=pl.ANY),
        ],
        out_specs=pl.BlockSpec(memory_space=pl.ANY),
        out_shape=jax.ShapeDtypeStruct((B, N), jnp.float32),
        scratch_shapes=[
            pltpu.VMEM((_NBUF, NW, nb, HID), jnp.float32),
            pltpu.VMEM((_NBUF, B, nb), jnp.float32),
            pltpu.SemaphoreType.DMA((_NBUF, NW)),
            pltpu.SemaphoreType.DMA((_NBUF, _RSPLIT)),
        ],
        compiler_params=pltpu.CompilerParams(
            dimension_semantics=("arbitrary",),
        ),
    )(hcat, bh, p, Wh)
    if tail:
        out = pl.pallas_call(
            _tail_body,
            grid=(1,),
            in_specs=[
                pl.BlockSpec((B, NW * HID), lambda n: (0, 0)),
                pl.BlockSpec((NW, nb, HID), lambda n: (0, n_full, 0)),
                pl.BlockSpec((NW, nb), lambda n: (0, n_full)),
                pl.BlockSpec((B, NW), lambda n: (0, 0)),
                pl.BlockSpec(memory_space=pl.ANY),
            ],
            out_specs=pl.BlockSpec((B, nb), lambda n: (0, n_full)),
            out_shape=jax.ShapeDtypeStruct((B, N), jnp.float32),
            input_output_aliases={4: 0},
        )(hcat, Wh, bh, p, out)
    return out


# speedup vs baseline: 1.1538x; 1.1538x over previous
import jax
import jax.numpy as jnp
from jax import lax
from jax.experimental import pallas as pl
from jax.experimental.pallas import tpu as pltpu

_NB = 1024
_NBUF = 4
_RSPLIT = 8


def _trunk_body(xs_ref, xm_ref, w1_ref, b1_ref, w2_ref, b2_ref, wld_ref,
                hcat_ref, p_ref):
    emb = xs_ref.shape[1]
    hid = w2_ref.shape[0]
    w1 = w1_ref[...]
    h = lax.dot_general(xs_ref[...], w1[:, :emb], (((1,), (1,)), ((), ())),
                        preferred_element_type=jnp.float32)
    h = h + lax.dot_general(xm_ref[...], w1[:, emb:], (((1,), (1,)), ((), ())),
                            preferred_element_type=jnp.float32)
    h = jnp.maximum(h + b1_ref[...], 0.0)
    h = lax.dot_general(h, w2_ref[...], (((1,), (1,)), ((), ())),
                        preferred_element_type=jnp.float32)
    h = jnp.maximum(h + b2_ref[...], 0.0)
    hb = h.astype(jnp.bfloat16)
    wld = wld_ref[...]
    nworlds = p_ref.shape[1]
    for w in range(nworlds):
        hcat_ref[:, w * hid:(w + 1) * hid] = jnp.where(wld == w, hb, 0)
    widx = lax.broadcasted_iota(jnp.int32, (wld.shape[0], nworlds), 1)
    p_ref[...] = (wld == widx).astype(jnp.bfloat16)


def _make_head_body(nb, n_full):
    def body(hcat_ref, bh_ref, p_ref, wh_hbm, out_hbm,
             wh_buf, acc_buf, sem_in, sem_out):
        nworlds = wh_hbm.shape[0]
        b = hcat_ref.shape[0]
        rs = b // _RSPLIT
        n = pl.program_id(0)
        nsteps = pl.num_programs(0)

        def read_copies(blk_idx, width):
            buf_slot = lax.rem(blk_idx, _NBUF)
            return [pltpu.make_async_copy(
                wh_hbm.at[w, pl.ds(blk_idx * nb, width), :],
                wh_buf.at[buf_slot, w, pl.ds(0, width), :],
                sem_in.at[buf_slot, w]) for w in range(nworlds)]

        def write_copies(blk_idx, width):
            buf_slot = lax.rem(blk_idx, _NBUF)
            return [pltpu.make_async_copy(
                acc_buf.at[buf_slot, pl.ds(r * rs, rs), pl.ds(0, width)],
                out_hbm.at[pl.ds(r * rs, rs), pl.ds(blk_idx * nb, width)],
                sem_out.at[buf_slot, r]) for r in range(_RSPLIT)]

        @pl.when(n == 0)
        def _():
            for m in range(min(_NBUF - 1, n_full)):
                for c in read_copies(m, nb):
                    c.start()

        @pl.when(n + _NBUF - 1 < n_full)
        def _():
            for c in read_copies(n + _NBUF - 1, nb):
                c.start()

        @pl.when(n >= _NBUF)
        def _():
            for c in write_copies(n - _NBUF, nb):
                c.wait()

        for c in read_copies(n, nb):
            c.wait()

        acc = lax.dot_general(p_ref[...], bh_ref[...].astype(jnp.bfloat16),
                              (((1,), (0,)), ((), ())),
                              preferred_element_type=jnp.float32)
        slot = lax.rem(n, _NBUF)
        whcat = jnp.concatenate(
            [wh_buf[slot, w] for w in range(nworlds)], axis=-1)
        acc = acc + lax.dot_general(hcat_ref[...], whcat,
                                    (((1,), (1,)), ((), ())),
                                    preferred_element_type=jnp.float32)
        acc_buf[slot] = acc.astype(jnp.bfloat16)

        for c in write_copies(n, nb):
            c.start()

        @pl.when(n == nsteps - 1)
        def _():
            for k in range(_NBUF - 1, -1, -1):
                @pl.when(n - k >= 0)
                def _(k=k):
                    for c in write_copies(n - k, nb):
                        c.wait()

    return body


def _tail_body(hcat_ref, wh_ref, bh_ref, p_ref, prev_ref, out_ref):
    nworlds = wh_ref.shape[0]
    acc = lax.dot_general(p_ref[...], bh_ref[...].astype(jnp.bfloat16),
                          (((1,), (0,)), ((), ())),
                          preferred_element_type=jnp.float32)
    whcat = jnp.concatenate(
        [wh_ref[w] for w in range(nworlds)], axis=-1)
    acc = acc + lax.dot_general(hcat_ref[...], whcat, (((1,), (1,)), ((), ())),
                                preferred_element_type=jnp.float32)
    out_ref[...] = acc.astype(jnp.bfloat16)


def kernel(source, mode, context_id, se, me, W1, b1, W2, b2, Wh, bh):
    B = source.shape[0]
    HID = W1.shape[0]
    NW, N, _ = Wh.shape

    xs = jnp.take(se, source, axis=0)
    xm = jnp.take(me, mode, axis=0)
    worlds = jnp.clip(context_id.astype(jnp.int32) - 1, 0, NW - 1)
    worlds = worlds.reshape(B, 1)

    hcat, p = pl.pallas_call(
        _trunk_body,
        out_shape=(jax.ShapeDtypeStruct((B, NW * HID), jnp.bfloat16),
                   jax.ShapeDtypeStruct((B, NW), jnp.bfloat16)),
    )(xs, xm, W1, b1.reshape(1, HID), W2, b2.reshape(1, HID), worlds)

    whb = Wh.astype(jnp.bfloat16)
    nb = _NB
    n_full = N // nb
    tail = N - n_full * nb
    out = pl.pallas_call(
        _make_head_body(nb, n_full),
        grid=(n_full,),
        in_specs=[
            pl.BlockSpec((B, NW * HID), lambda n: (0, 0)),
            pl.BlockSpec((NW, nb), lambda n: (0, n)),
            pl.BlockSpec((B, NW), lambda n: (0, 0)),
            pl.BlockSpec(memory_space=pl.ANY),
        ],
        out_specs=pl.BlockSpec(memory_space=pl.ANY),
        out_shape=jax.ShapeDtypeStruct((B, N), jnp.bfloat16),
        scratch_shapes=[
            pltpu.VMEM((_NBUF, NW, nb, HID), jnp.bfloat16),
            pltpu.VMEM((_NBUF, B, nb), jnp.bfloat16),
            pltpu.SemaphoreType.DMA((_NBUF, NW)),
            pltpu.SemaphoreType.DMA((_NBUF, _RSPLIT)),
        ],
        compiler_params=pltpu.CompilerParams(
            dimension_semantics=("arbitrary",),
        ),
    )(hcat, bh, p, whb)
    if tail:
        out = pl.pallas_call(
            _tail_body,
            grid=(1,),
            in_specs=[
                pl.BlockSpec((B, NW * HID), lambda n: (0, 0)),
                pl.BlockSpec((NW, nb, HID), lambda n: (0, n_full, 0)),
                pl.BlockSpec((NW, nb), lambda n: (0, n_full)),
                pl.BlockSpec((B, NW), lambda n: (0, 0)),
                pl.BlockSpec(memory_space=pl.ANY),
            ],
            out_specs=pl.BlockSpec((B, nb), lambda n: (0, n_full)),
            out_shape=jax.ShapeDtypeStruct((B, N), jnp.bfloat16),
            input_output_aliases={4: 0},
        )(hcat, whb, bh, p, out)
    return out.astype(jnp.float32)
